# baseline (speedup 1.0000x reference)
import functools

import jax
import jax.numpy as jnp
from jax import lax
from jax.experimental import pallas as pl
from jax.experimental.pallas import tpu as pltpu
from jax.experimental.pallas import tpu_sc as plsc

B, L, H, HEADS, NE, M = 4, 512, 768, 12, 42, 3
EMB = 768
R = NE * NE
NEP = 48
RP = NEP * NEP
JJ = 48
QD = NE * JJ
QTOT = B * QD
G = 6
TI = NE // G
QROWS = TI * JJ

NW = 32
BPW = 256
CH = 64
NCH = BPW // CH


def _k1_body(idx_ref, seq_ref, att_ref, wht_ref, wtt_ref, bh_ref, bt_ref,
             pn_ref, eh_ref, et_ref, p_scr):
    seq = seq_ref[0]
    idx = idx_ref[0]
    iota = lax.broadcasted_iota(jnp.int32, (NEP, L), 1)
    oh = [(idx[:, m:m + 1] == iota).astype(jnp.float32) for m in range(M)]
    es = [jnp.dot(o, seq, preferred_element_type=jnp.float32) for o in oh]
    mx = jnp.maximum(jnp.maximum(es[0], es[1]), es[2])
    ee = mx + jnp.log(jnp.exp(es[0] - mx) + jnp.exp(es[1] - mx)
                      + jnp.exp(es[2] - mx))
    eh_ref[0] = (jnp.dot(ee, wht_ref[...], preferred_element_type=jnp.float32)
                 + bh_ref[...])
    et_ref[0] = (jnp.dot(ee, wtt_ref[...], preferred_element_type=jnp.float32)
                 + bt_ref[...])
    g = (oh[0] + oh[1] + oh[2]) * (1.0 / M)
    for h in range(HEADS):
        ea = jnp.dot(g, att_ref[0, h], preferred_element_type=jnp.float32)
        contrib = ea[:, None, :] * ea[None, :, :]
        if h == 0:
            p_scr[...] = contrib
        else:
            p_scr[...] += contrib
    p = jnp.maximum(p_scr[...] * (1.0 / HEADS), 0.0)
    s = jnp.sum(p, axis=-1, keepdims=True)
    pn_ref[0] = (p / (s + 1e-10)).reshape(RP, L)


@functools.cache
def _make_pn_gather():
    @functools.partial(
        pl.kernel,
        mesh=plsc.VectorSubcoreMesh(core_axis_name="c", subcore_axis_name="s"),
        out_type=jax.ShapeDtypeStruct((QTOT, L), jnp.float32),
        scratch_types=[
            pltpu.VMEM((NCH, CH), jnp.int32),
            pltpu.VMEM((CH, L), jnp.float32),
            pltpu.VMEM((CH, L), jnp.float32),
            pltpu.SemaphoreType.DMA,
            pltpu.SemaphoreType.DMA,
        ],
    )
    def pn_gather(tab, gidx, out, idx_v, r0_v, r1_v, s0, s1):
        wid = lax.axis_index("s") * 2 + lax.axis_index("c")
        pltpu.sync_copy(gidx.at[wid], idx_v)
        bufs = [r0_v, r1_v]
        sems = [s0, s1]

        def start(c):
            @pl.when(wid * BPW + (c + 1) * CH <= QTOT)
            def _():
                pltpu.async_copy(tab.at[idx_v.at[c]], bufs[c % 2],
                                 sems[c % 2])

        def finish(c):
            @pl.when(wid * BPW + (c + 1) * CH <= QTOT)
            def _():
                pltpu.make_async_copy(tab.at[idx_v.at[c]], bufs[c % 2],
                                      sems[c % 2]).wait()
                pltpu.sync_copy(bufs[c % 2],
                                out.at[pl.ds(wid * BPW + c * CH, CH)])

        start(0)
        start(1)
        for c in range(NCH):
            finish(c)
            if c + 2 < NCH:
                start(c + 2)

    return pn_gather


def _pn_gather(tab, gidx):
    return _make_pn_gather()(tab, gidx)


def _k3_body(png_ref, seq_ref, ee_ref, wb_ref, q_ref, out_ref):
    rows = png_ref[0].reshape(QROWS, L)
    rs = jnp.dot(rows.astype(jnp.bfloat16), seq_ref[0].astype(jnp.bfloat16),
                 preferred_element_type=jnp.float32)
    z = jnp.dot(rs.astype(jnp.bfloat16), wb_ref[...].astype(jnp.bfloat16),
                preferred_element_type=jnp.float32)
    iota = lax.broadcasted_iota(jnp.int32, (NEP, QROWS), 0)
    ohq = (iota == q_ref[0, 0]).astype(jnp.float32)
    dn = (((0,), (0,)), ((), ()))
    eg = lax.dot_general(ohq, ee_ref[0], dn,
                         preferred_element_type=jnp.float32)
    t = jnp.tanh(z + eg).reshape(TI, JJ, EMB)
    out_ref[0] = t[:, :NE, :]


def kernel(sequence_output, attention, mention_starts, hts, W_h, b_h, W_t, b_t):
    idx = (mention_starts + 1).astype(jnp.int32)
    idx = jnp.pad(idx, ((0, 0), (0, NEP - NE), (0, 0)), constant_values=-1)
    bh2 = b_h.reshape(1, EMB)
    bt2 = b_t.reshape(1, EMB)

    pn, eh, et = pl.pallas_call(
        _k1_body,
        grid=(B,),
        in_specs=[
            pl.BlockSpec((1, NEP, M), lambda b: (b, 0, 0)),
            pl.BlockSpec((1, L, H), lambda b: (b, 0, 0)),
            pl.BlockSpec((1, HEADS, L, L), lambda b: (b, 0, 0, 0)),
            pl.BlockSpec((H, EMB), lambda b: (0, 0)),
            pl.BlockSpec((H, EMB), lambda b: (0, 0)),
            pl.BlockSpec((1, EMB), lambda b: (0, 0)),
            pl.BlockSpec((1, EMB), lambda b: (0, 0)),
        ],
        out_specs=[
            pl.BlockSpec((1, RP, L), lambda b: (b, 0, 0)),
            pl.BlockSpec((1, NEP, EMB), lambda b: (b, 0, 0)),
            pl.BlockSpec((1, NEP, EMB), lambda b: (b, 0, 0)),
        ],
        out_shape=[
            jax.ShapeDtypeStruct((B, RP, L), jnp.float32),
            jax.ShapeDtypeStruct((B, NEP, EMB), jnp.float32),
            jax.ShapeDtypeStruct((B, NEP, EMB), jnp.float32),
        ],
        scratch_shapes=[pltpu.VMEM((NEP, NEP, L), jnp.float32)],
    )(idx, sequence_output, attention, W_h, W_t, bh2, bt2)

    h2 = hts.astype(jnp.int32).reshape(B, NE, NE, 2)
    hq = jnp.pad(h2[..., 0], ((0, 0), (0, 0), (0, JJ - NE)))
    tq = jnp.pad(h2[..., 1], ((0, 0), (0, 0), (0, JJ - NE)))
    gq = (jnp.arange(B, dtype=jnp.int32)[:, None, None] * RP
          + hq * NEP + tq).reshape(-1)
    gq = jnp.concatenate(
        [gq, jnp.zeros((NW * BPW - QTOT,), jnp.int32)]).reshape(NW, NCH, CH)

    png = _pn_gather(pn.reshape(B * RP, L), gq)

    png4 = png.reshape(B, NE, JJ, L)
    hqr = hq.reshape(B, G, 1, QROWS)
    tqr = tq.reshape(B, G, 1, QROWS)

    def extractor(ee, w, qr):
        return pl.pallas_call(
            _k3_body,
            grid=(B, G),
            in_specs=[
                pl.BlockSpec((1, TI, JJ, L), lambda b, t: (b, t, 0, 0)),
                pl.BlockSpec((1, L, H), lambda b, t: (b, 0, 0)),
                pl.BlockSpec((1, NEP, EMB), lambda b, t: (b, 0, 0)),
                pl.BlockSpec((H, EMB), lambda b, t: (1, 0)),
                pl.BlockSpec((1, 1, 1, QROWS), lambda b, t: (b, t, 0, 0)),
            ],
            out_specs=pl.BlockSpec((1, TI, NE, EMB), lambda b, t: (b, t, 0, 0)),
            out_shape=jax.ShapeDtypeStruct((B, NE, NE, EMB), jnp.float32),
        )(png4, sequence_output, ee, w, qr)

    outh = extractor(eh, W_h, hqr)
    outt = extractor(et, W_t, tqr)
    return (outh, outt)


# speedup vs baseline: 1.1910x; 1.1910x over previous
import functools

import jax
import jax.numpy as jnp
from jax import lax
from jax.experimental import pallas as pl
from jax.experimental.pallas import tpu as pltpu
from jax.experimental.pallas import tpu_sc as plsc

B, L, H, HEADS, NE, M = 4, 512, 768, 12, 42, 3
EMB = 768
R = NE * NE
NEP = 48
RP = NEP * NEP
JJ = 48
QD = NE * JJ
QTOT = B * QD
G = 6
TI = NE // G
QROWS = TI * JJ

NW = 32
BPW = 256
CH = 64
NCH = BPW // CH


def _k1_body(idx_ref, seq_ref, att_ref, wht_ref, wtt_ref, bh_ref, bt_ref,
             pn_ref, eh_ref, et_ref, p_scr):
    seq = seq_ref[0]
    idx = idx_ref[0]
    iota = lax.broadcasted_iota(jnp.int32, (NEP, L), 1)
    oh = [(idx[:, m:m + 1] == iota).astype(jnp.float32) for m in range(M)]
    es = [jnp.dot(o, seq, preferred_element_type=jnp.float32) for o in oh]
    mx = jnp.maximum(jnp.maximum(es[0], es[1]), es[2])
    ee = mx + jnp.log(jnp.exp(es[0] - mx) + jnp.exp(es[1] - mx)
                      + jnp.exp(es[2] - mx))
    eh_ref[0] = (jnp.dot(ee, wht_ref[...], preferred_element_type=jnp.float32)
                 + bh_ref[...])
    et_ref[0] = (jnp.dot(ee, wtt_ref[...], preferred_element_type=jnp.float32)
                 + bt_ref[...])
    g = (oh[0] + oh[1] + oh[2]) * (1.0 / M)
    for h in range(HEADS):
        ea = jnp.dot(g, att_ref[0, h], preferred_element_type=jnp.float32)
        contrib = ea[:, None, :] * ea[None, :, :]
        if h == 0:
            p_scr[...] = contrib
        else:
            p_scr[...] += contrib
    p = jnp.maximum(p_scr[...] * (1.0 / HEADS), 0.0)
    s = jnp.sum(p, axis=-1, keepdims=True)
    pn_ref[0] = (p / (s + 1e-10)).reshape(RP, L)


@functools.cache
def _make_pn_gather():
    @functools.partial(
        pl.kernel,
        mesh=plsc.VectorSubcoreMesh(core_axis_name="c", subcore_axis_name="s"),
        out_type=jax.ShapeDtypeStruct((QTOT, L), jnp.float32),
        scratch_types=[
            pltpu.VMEM((NCH, CH), jnp.int32),
            pltpu.VMEM((CH, L), jnp.float32),
            pltpu.VMEM((CH, L), jnp.float32),
            pltpu.SemaphoreType.DMA,
            pltpu.SemaphoreType.DMA,
        ],
    )
    def pn_gather(tab, gidx, out, idx_v, r0_v, r1_v, s0, s1):
        wid = lax.axis_index("s") * 2 + lax.axis_index("c")
        pltpu.sync_copy(gidx.at[wid], idx_v)
        bufs = [r0_v, r1_v]
        sems = [s0, s1]

        def start(c):
            @pl.when(wid * BPW + (c + 1) * CH <= QTOT)
            def _():
                pltpu.async_copy(tab.at[idx_v.at[c]], bufs[c % 2],
                                 sems[c % 2])

        def finish(c):
            @pl.when(wid * BPW + (c + 1) * CH <= QTOT)
            def _():
                pltpu.make_async_copy(tab.at[idx_v.at[c]], bufs[c % 2],
                                      sems[c % 2]).wait()
                pltpu.sync_copy(bufs[c % 2],
                                out.at[pl.ds(wid * BPW + c * CH, CH)])

        start(0)
        start(1)
        for c in range(NCH):
            finish(c)
            if c + 2 < NCH:
                start(c + 2)

    return pn_gather


def _pn_gather(tab, gidx):
    return _make_pn_gather()(tab, gidx)


def _k3_body(png_ref, seq_ref, eh_ref, et_ref, whb_ref, wtb_ref,
             hq_ref, tq_ref, outh_ref, outt_ref):
    rows = png_ref[0].reshape(QROWS, L)
    rs = jnp.dot(rows.astype(jnp.bfloat16), seq_ref[0].astype(jnp.bfloat16),
                 preferred_element_type=jnp.float32)
    rsb = rs.astype(jnp.bfloat16)
    zh = jnp.dot(rsb, whb_ref[...].astype(jnp.bfloat16),
                 preferred_element_type=jnp.float32)
    zt = jnp.dot(rsb, wtb_ref[...].astype(jnp.bfloat16),
                 preferred_element_type=jnp.float32)
    iota = lax.broadcasted_iota(jnp.int32, (NEP, QROWS), 0)
    ohh = (iota == hq_ref[0, 0]).astype(jnp.float32)
    oht = (iota == tq_ref[0, 0]).astype(jnp.float32)
    dn = (((0,), (0,)), ((), ()))
    ehg = lax.dot_general(ohh, eh_ref[0], dn,
                          preferred_element_type=jnp.float32)
    etg = lax.dot_general(oht, et_ref[0], dn,
                          preferred_element_type=jnp.float32)
    th = jnp.tanh(zh + ehg).reshape(TI, JJ, EMB)
    tt = jnp.tanh(zt + etg).reshape(TI, JJ, EMB)
    outh_ref[0] = th[:, :NE, :]
    outt_ref[0] = tt[:, :NE, :]


def kernel(sequence_output, attention, mention_starts, hts, W_h, b_h, W_t, b_t):
    idx = (mention_starts + 1).astype(jnp.int32)
    idx = jnp.pad(idx, ((0, 0), (0, NEP - NE), (0, 0)), constant_values=-1)
    bh2 = b_h.reshape(1, EMB)
    bt2 = b_t.reshape(1, EMB)

    pn, eh, et = pl.pallas_call(
        _k1_body,
        grid=(B,),
        in_specs=[
            pl.BlockSpec((1, NEP, M), lambda b: (b, 0, 0)),
            pl.BlockSpec((1, L, H), lambda b: (b, 0, 0)),
            pl.BlockSpec((1, HEADS, L, L), lambda b: (b, 0, 0, 0)),
            pl.BlockSpec((H, EMB), lambda b: (0, 0)),
            pl.BlockSpec((H, EMB), lambda b: (0, 0)),
            pl.BlockSpec((1, EMB), lambda b: (0, 0)),
            pl.BlockSpec((1, EMB), lambda b: (0, 0)),
        ],
        out_specs=[
            pl.BlockSpec((1, RP, L), lambda b: (b, 0, 0)),
            pl.BlockSpec((1, NEP, EMB), lambda b: (b, 0, 0)),
            pl.BlockSpec((1, NEP, EMB), lambda b: (b, 0, 0)),
        ],
        out_shape=[
            jax.ShapeDtypeStruct((B, RP, L), jnp.float32),
            jax.ShapeDtypeStruct((B, NEP, EMB), jnp.float32),
            jax.ShapeDtypeStruct((B, NEP, EMB), jnp.float32),
        ],
        scratch_shapes=[pltpu.VMEM((NEP, NEP, L), jnp.float32)],
    )(idx, sequence_output, attention, W_h, W_t, bh2, bt2)

    h2 = hts.astype(jnp.int32).reshape(B, NE, NE, 2)
    hq = jnp.pad(h2[..., 0], ((0, 0), (0, 0), (0, JJ - NE)))
    tq = jnp.pad(h2[..., 1], ((0, 0), (0, 0), (0, JJ - NE)))
    gq = (jnp.arange(B, dtype=jnp.int32)[:, None, None] * RP
          + hq * NEP + tq).reshape(-1)
    gq = jnp.concatenate(
        [gq, jnp.zeros((NW * BPW - QTOT,), jnp.int32)]).reshape(NW, NCH, CH)

    png = _pn_gather(pn.reshape(B * RP, L), gq)

    png4 = png.reshape(B, NE, JJ, L)
    hqr = hq.reshape(B, G, 1, QROWS)
    tqr = tq.reshape(B, G, 1, QROWS)

    outh, outt = pl.pallas_call(
        _k3_body,
        grid=(B, G),
        in_specs=[
            pl.BlockSpec((1, TI, JJ, L), lambda b, t: (b, t, 0, 0)),
            pl.BlockSpec((1, L, H), lambda b, t: (b, 0, 0)),
            pl.BlockSpec((1, NEP, EMB), lambda b, t: (b, 0, 0)),
            pl.BlockSpec((1, NEP, EMB), lambda b, t: (b, 0, 0)),
            pl.BlockSpec((H, EMB), lambda b, t: (1, 0)),
            pl.BlockSpec((H, EMB), lambda b, t: (1, 0)),
            pl.BlockSpec((1, 1, 1, QROWS), lambda b, t: (b, t, 0, 0)),
            pl.BlockSpec((1, 1, 1, QROWS), lambda b, t: (b, t, 0, 0)),
        ],
        out_specs=[
            pl.BlockSpec((1, TI, NE, EMB), lambda b, t: (b, t, 0, 0)),
            pl.BlockSpec((1, TI, NE, EMB), lambda b, t: (b, t, 0, 0)),
        ],
        out_shape=[
            jax.ShapeDtypeStruct((B, NE, NE, EMB), jnp.float32),
            jax.ShapeDtypeStruct((B, NE, NE, EMB), jnp.float32),
        ],
    )(png4, sequence_output, eh, et, W_h, W_t, hqr, tqr)
    return (outh, outt)


# speedup vs baseline: 1.1990x; 1.0067x over previous
import functools

import jax
import jax.numpy as jnp
from jax import lax
from jax.experimental import pallas as pl
from jax.experimental.pallas import tpu as pltpu
from jax.experimental.pallas import tpu_sc as plsc

B, L, H, HEADS, NE, M = 4, 512, 768, 12, 42, 3
EMB = 768
R = NE * NE
NEP = 48
RP = NEP * NEP
JJ = 48
QD = NE * JJ
QTOT = B * QD
G = 6
TI = NE // G
QROWS = TI * JJ

NW = 32
BPW = 256
CH = 64
NCH = BPW // CH


def _k1_body(idx_ref, seq_ref, att_ref, wht_ref, wtt_ref, bh_ref, bt_ref,
             pn_ref, eh_ref, et_ref, p_scr):
    seq = seq_ref[0]
    idx = idx_ref[0]
    iota = lax.broadcasted_iota(jnp.int32, (NEP, L), 1)
    oh = [(idx[:, m:m + 1] == iota).astype(jnp.float32) for m in range(M)]
    es = [jnp.dot(o, seq, preferred_element_type=jnp.float32) for o in oh]
    mx = jnp.maximum(jnp.maximum(es[0], es[1]), es[2])
    ee = mx + jnp.log(jnp.exp(es[0] - mx) + jnp.exp(es[1] - mx)
                      + jnp.exp(es[2] - mx))
    eh_ref[0] = (jnp.dot(ee, wht_ref[...], preferred_element_type=jnp.float32)
                 + bh_ref[...])
    et_ref[0] = (jnp.dot(ee, wtt_ref[...], preferred_element_type=jnp.float32)
                 + bt_ref[...])
    g = (oh[0] + oh[1] + oh[2]) * (1.0 / M)
    for h in range(HEADS):
        ea = jnp.dot(g, att_ref[0, h], preferred_element_type=jnp.float32)
        contrib = ea[:, None, :] * ea[None, :, :]
        if h == 0:
            p_scr[...] = contrib
        else:
            p_scr[...] += contrib
    p = jnp.maximum(p_scr[...] * (1.0 / HEADS), 0.0)
    s = jnp.sum(p, axis=-1, keepdims=True)
    pn_ref[0] = (p / (s + 1e-10)).reshape(RP, L)


@functools.cache
def _make_pn_gather():
    @functools.partial(
        pl.kernel,
        mesh=plsc.VectorSubcoreMesh(core_axis_name="c", subcore_axis_name="s"),
        out_type=jax.ShapeDtypeStruct((QTOT, L), jnp.float32),
        scratch_types=[
            pltpu.VMEM((NCH, CH), jnp.int32),
            pltpu.VMEM((CH, L), jnp.float32),
            pltpu.VMEM((CH, L), jnp.float32),
            pltpu.VMEM((CH, L), jnp.float32),
            pltpu.SemaphoreType.DMA,
            pltpu.SemaphoreType.DMA,
            pltpu.SemaphoreType.DMA,
        ],
    )
    def pn_gather(tab, gidx, out, idx_v, r0_v, r1_v, r2_v, s0, s1, s2):
        wid = lax.axis_index("s") * 2 + lax.axis_index("c")
        pltpu.sync_copy(gidx.at[wid], idx_v)
        bufs = [r0_v, r1_v, r2_v]
        sems = [s0, s1, s2]

        def start(c):
            @pl.when(wid * BPW + (c + 1) * CH <= QTOT)
            def _():
                pltpu.async_copy(tab.at[idx_v.at[c]], bufs[c % 3],
                                 sems[c % 3])

        def finish(c):
            @pl.when(wid * BPW + (c + 1) * CH <= QTOT)
            def _():
                pltpu.make_async_copy(tab.at[idx_v.at[c]], bufs[c % 3],
                                      sems[c % 3]).wait()
                pltpu.sync_copy(bufs[c % 3],
                                out.at[pl.ds(wid * BPW + c * CH, CH)])

        start(0)
        start(1)
        start(2)
        for c in range(NCH):
            finish(c)
            if c + 3 < NCH:
                start(c + 3)

    return pn_gather


def _pn_gather(tab, gidx):
    return _make_pn_gather()(tab, gidx)


def _k3_body(png_ref, seq_ref, eh_ref, et_ref, whb_ref, wtb_ref,
             hq_ref, tq_ref, outh_ref, outt_ref):
    rows = png_ref[0].reshape(QROWS, L)
    rs = jnp.dot(rows.astype(jnp.bfloat16), seq_ref[0].astype(jnp.bfloat16),
                 preferred_element_type=jnp.float32)
    rsb = rs.astype(jnp.bfloat16)
    zh = jnp.dot(rsb, whb_ref[...].astype(jnp.bfloat16),
                 preferred_element_type=jnp.float32)
    zt = jnp.dot(rsb, wtb_ref[...].astype(jnp.bfloat16),
                 preferred_element_type=jnp.float32)
    iota = lax.broadcasted_iota(jnp.int32, (NEP, QROWS), 0)
    ohh = (iota == hq_ref[0, 0]).astype(jnp.float32)
    oht = (iota == tq_ref[0, 0]).astype(jnp.float32)
    dn = (((0,), (0,)), ((), ()))
    ehg = lax.dot_general(ohh, eh_ref[0], dn,
                          preferred_element_type=jnp.float32)
    etg = lax.dot_general(oht, et_ref[0], dn,
                          preferred_element_type=jnp.float32)
    th = jnp.tanh(zh + ehg).reshape(TI, JJ, EMB)
    tt = jnp.tanh(zt + etg).reshape(TI, JJ, EMB)
    outh_ref[0] = th[:, :NE, :]
    outt_ref[0] = tt[:, :NE, :]


def kernel(sequence_output, attention, mention_starts, hts, W_h, b_h, W_t, b_t):
    idx = (mention_starts + 1).astype(jnp.int32)
    idx = jnp.pad(idx, ((0, 0), (0, NEP - NE), (0, 0)), constant_values=-1)
    bh2 = b_h.reshape(1, EMB)
    bt2 = b_t.reshape(1, EMB)

    pn, eh, et = pl.pallas_call(
        _k1_body,
        grid=(B,),
        in_specs=[
            pl.BlockSpec((1, NEP, M), lambda b: (b, 0, 0)),
            pl.BlockSpec((1, L, H), lambda b: (b, 0, 0)),
            pl.BlockSpec((1, HEADS, L, L), lambda b: (b, 0, 0, 0)),
            pl.BlockSpec((H, EMB), lambda b: (0, 0)),
            pl.BlockSpec((H, EMB), lambda b: (0, 0)),
            pl.BlockSpec((1, EMB), lambda b: (0, 0)),
            pl.BlockSpec((1, EMB), lambda b: (0, 0)),
        ],
        out_specs=[
            pl.BlockSpec((1, RP, L), lambda b: (b, 0, 0)),
            pl.BlockSpec((1, NEP, EMB), lambda b: (b, 0, 0)),
            pl.BlockSpec((1, NEP, EMB), lambda b: (b, 0, 0)),
        ],
        out_shape=[
            jax.ShapeDtypeStruct((B, RP, L), jnp.float32),
            jax.ShapeDtypeStruct((B, NEP, EMB), jnp.float32),
            jax.ShapeDtypeStruct((B, NEP, EMB), jnp.float32),
        ],
        scratch_shapes=[pltpu.VMEM((NEP, NEP, L), jnp.float32)],
    )(idx, sequence_output, attention, W_h, W_t, bh2, bt2)

    h2 = hts.astype(jnp.int32).reshape(B, NE, NE, 2)
    hq = jnp.pad(h2[..., 0], ((0, 0), (0, 0), (0, JJ - NE)))
    tq = jnp.pad(h2[..., 1], ((0, 0), (0, 0), (0, JJ - NE)))
    gq = (jnp.arange(B, dtype=jnp.int32)[:, None, None] * RP
          + hq * NEP + tq).reshape(-1)
    gq = jnp.concatenate(
        [gq, jnp.zeros((NW * BPW - QTOT,), jnp.int32)]).reshape(NW, NCH, CH)

    png = _pn_gather(pn.reshape(B * RP, L), gq)

    png4 = png.reshape(B, NE, JJ, L)
    hqr = hq.reshape(B, G, 1, QROWS)
    tqr = tq.reshape(B, G, 1, QROWS)

    outh, outt = pl.pallas_call(
        _k3_body,
        grid=(B, G),
        in_specs=[
            pl.BlockSpec((1, TI, JJ, L), lambda b, t: (b, t, 0, 0)),
            pl.BlockSpec((1, L, H), lambda b, t: (b, 0, 0)),
            pl.BlockSpec((1, NEP, EMB), lambda b, t: (b, 0, 0)),
            pl.BlockSpec((1, NEP, EMB), lambda b, t: (b, 0, 0)),
            pl.BlockSpec((H, EMB), lambda b, t: (1, 0)),
            pl.BlockSpec((H, EMB), lambda b, t: (1, 0)),
            pl.BlockSpec((1, 1, 1, QROWS), lambda b, t: (b, t, 0, 0)),
            pl.BlockSpec((1, 1, 1, QROWS), lambda b, t: (b, t, 0, 0)),
        ],
        out_specs=[
            pl.BlockSpec((1, TI, NE, EMB), lambda b, t: (b, t, 0, 0)),
            pl.BlockSpec((1, TI, NE, EMB), lambda b, t: (b, t, 0, 0)),
        ],
        out_shape=[
            jax.ShapeDtypeStruct((B, NE, NE, EMB), jnp.float32),
            jax.ShapeDtypeStruct((B, NE, NE, EMB), jnp.float32),
        ],
    )(png4, sequence_output, eh, et, W_h, W_t, hqr, tqr)
    return (outh, outt)
